# baseline (speedup 1.0000x reference)
import jax
import jax.numpy as jnp
from jax import lax
from jax.experimental import pallas as pl
from jax.experimental.pallas import tpu as pltpu
from jax.experimental.pallas import tpu_sc as plsc

NF = 26
VOCAB = 100000
D = 32
B = 16384
NC, NS, L = 2, 16, 16
NW = NC * NS

VC = 640
K_PER_F = (VOCAB // VC)
TAIL_V = VOCAB - K_PER_F * VC
PROWS = VC * D // 128
N_CHUNKS = NF * K_PER_F
K1_ITERS = -(-N_CHUNKS // NW)

CHUNK = 128
N_BCHUNKS = B // CHUNK
TOT_CHUNKS = NF * N_BCHUNKS
CPW = TOT_CHUNKS // NW
OP = CHUNK + 9


RB = 320
PB = RB * D // 128
N_DCH = NF * VOCAB // RB
K1B_ITERS = -(-N_DCH // NW)


def _detile(d2):
    def body(t_hbm, pk_hbm, st0, st1, pack0, pack1, ss0, ss1, ws0, ws1):
        wid = lax.axis_index("s") * NC + lax.axis_index("c")

        def v0_of(i):
            return jnp.minimum(wid + NW * i, N_DCH - 1) * RB

        def fire_st(i, st, sem):
            pltpu.async_copy(
                t_hbm.at[pl.ds(pl.multiple_of(v0_of(i), 8), RB)], st, sem)

        def wait_st(st, sem):
            pltpu.make_async_copy(t_hbm.at[pl.ds(0, RB)], st, sem).wait()

        def repack(st, pack):
            def prow(q2, carry):
                for u in range(2):
                    q = q2 * 2 + u
                    for vi in range(4):
                        for h in range(2):
                            pack[q, pl.ds(vi * D + h * L, L)] = (
                                st[q * 4 + vi, pl.ds(h * L, L)])
                return carry
            lax.fori_loop(0, PB // 2, prow, 0)

        def fire_pack(i, pack, sem):
            r0 = pl.multiple_of(v0_of(i) // 4, 8)
            pltpu.async_copy(pack, pk_hbm.at[pl.ds(r0, PB)], sem)

        def wait_pack(pack, sem):
            pltpu.make_async_copy(pack, pk_hbm.at[pl.ds(0, PB)], sem).wait()

        fire_st(0, st0, ss0)

        def gloop(i, carry):
            even = (i % 2) == 0

            @pl.when(even)
            def _():
                wait_st(st0, ss0)
                @pl.when(i + 1 < K1B_ITERS)
                def _():
                    fire_st(i + 1, st1, ss1)
                @pl.when(i >= 2)
                def _():
                    wait_pack(pack0, ws0)
                repack(st0, pack0)
                fire_pack(i, pack0, ws0)

            @pl.when(jnp.logical_not(even))
            def _():
                wait_st(st1, ss1)
                @pl.when(i + 1 < K1B_ITERS)
                def _():
                    fire_st(i + 1, st0, ss0)
                @pl.when(i >= 2)
                def _():
                    wait_pack(pack1, ws1)
                repack(st1, pack1)
                fire_pack(i, pack1, ws1)

            return carry

        lax.fori_loop(0, K1B_ITERS, gloop, 0)

        wait_pack(pack0, ws0)
        wait_pack(pack1, ws1)

    return pl.kernel(
        body,
        out_type=jax.ShapeDtypeStruct((NF * VOCAB * D // 128, 128),
                                      jnp.float32),
        mesh=plsc.VectorSubcoreMesh(
            core_axis_name="c", subcore_axis_name="s",
            num_cores=NC, num_subcores=NS,
        ),
        scratch_types=[
            pltpu.VMEM((RB, D), jnp.float32),
            pltpu.VMEM((RB, D), jnp.float32),
            pltpu.VMEM((PB, 128), jnp.float32),
            pltpu.VMEM((PB, 128), jnp.float32),
            pltpu.SemaphoreType.DMA,
            pltpu.SemaphoreType.DMA,
            pltpu.SemaphoreType.DMA,
            pltpu.SemaphoreType.DMA,
        ],
        compiler_params=pltpu.CompilerParams(use_tc_tiling_on_sc=True,
                                             needs_layout_passes=False),
    )(d2)


def _gather(pk2d, xf):
    def body(tab, xf_hbm, out_hbm, idx_v, rows0, rows1, o0, o1,
             gs0, gs1, ws0, ws1):
        wid = lax.axis_index("s") * NC + lax.axis_index("c")
        c0 = wid * CPW
        lanes = lax.iota(jnp.int32, L)

        pltpu.sync_copy(xf_hbm.at[pl.ds(c0, CPW)], idx_v)

        def ixrow(cc, carry):
            f = (c0 + cc) // N_BCHUNKS
            for j in range(CHUNK // L):
                sl = pl.ds(j * L, L)
                idx_v[cc, sl] = idx_v[cc, sl] + f * VOCAB
            return carry
        lax.fori_loop(0, CPW, ixrow, 0)

        def fire_gather(cc, rows, sem):
            pltpu.async_copy(tab.at[idx_v.at[cc]], rows, sem)

        def wait_gather(rows, sem):
            pltpu.make_async_copy(tab.at[pl.ds(0, CHUNK)], rows, sem).wait()

        def transpose(rows, o):
            rv0 = lanes
            rv1 = 16 + lanes

            def irow(i8, carry):
                for u in range(8):
                    i = i8 * 8 + u
                    cv = lanes * 0 + i
                    plsc.store_scatter(o, [rv0, cv], rows[i, pl.ds(0, 16)])
                    plsc.store_scatter(o, [rv1, cv], rows[i, pl.ds(16, 16)])
                return carry
            lax.fori_loop(0, CHUNK // 8, irow, 0)

        def fire_writes(cc, o, sem):
            c = c0 + cc
            f = c // N_BCHUNKS
            bt = c - f * N_BCHUNKS
            for dt in range(4):
                pltpu.async_copy(o.at[pl.ds(dt * 8, 8), pl.ds(0, CHUNK)],
                                 out_hbm.at[f, dt, bt], sem)

        def wait_writes(o, sem):
            for dt in range(4):
                pltpu.make_async_copy(o.at[pl.ds(0, 8), pl.ds(0, CHUNK)],
                                      out_hbm.at[0, 0, 0], sem).wait()

        fire_gather(0, rows0, gs0)

        def gloop(cc, carry):
            even = (cc % 2) == 0

            @pl.when(even)
            def _():
                wait_gather(rows0, gs0)
                @pl.when(cc + 1 < CPW)
                def _():
                    fire_gather(cc + 1, rows1, gs1)
                @pl.when(cc >= 2)
                def _():
                    wait_writes(o0, ws0)
                transpose(rows0, o0)
                fire_writes(cc, o0, ws0)

            @pl.when(jnp.logical_not(even))
            def _():
                wait_gather(rows1, gs1)
                @pl.when(cc + 1 < CPW)
                def _():
                    fire_gather(cc + 1, rows0, gs0)
                @pl.when(cc >= 2)
                def _():
                    wait_writes(o1, ws1)
                transpose(rows1, o1)
                fire_writes(cc, o1, ws1)

            return carry

        lax.fori_loop(0, CPW, gloop, 0)

        wait_writes(o0, ws0)
        wait_writes(o1, ws1)

    return pl.kernel(
        body,
        out_type=jax.ShapeDtypeStruct((NF, 4, N_BCHUNKS, 8, 128),
                                      jnp.float32),
        mesh=plsc.VectorSubcoreMesh(
            core_axis_name="c", subcore_axis_name="s",
            num_cores=NC, num_subcores=NS,
        ),
        scratch_types=[
            pltpu.VMEM((CPW, CHUNK), jnp.int32),
            pltpu.VMEM((CHUNK, D), jnp.float32),
            pltpu.VMEM((CHUNK, D), jnp.float32),
            pltpu.VMEM((D, OP), jnp.float32),
            pltpu.VMEM((D, OP), jnp.float32),
            pltpu.SemaphoreType.DMA,
            pltpu.SemaphoreType.DMA,
            pltpu.SemaphoreType.DMA,
            pltpu.SemaphoreType.DMA,
        ],
        compiler_params=pltpu.CompilerParams(use_tc_tiling_on_sc=False,
                                             needs_layout_passes=False),
    )(pk2d, xf)


def kernel(x, tables):
    b, nf = x.shape
    _, vocab, d = tables.shape
    xf = jnp.transpose(x.astype(jnp.int32)).reshape(nf * b // CHUNK, CHUNK)
    pk = _detile(tables.reshape(nf * vocab, d))
    out5 = _gather(pk.reshape(nf * vocab, d), xf)
    return jnp.transpose(out5, (2, 4, 0, 1, 3)).reshape(b, nf, d)


# speedup vs baseline: 1.0010x; 1.0010x over previous
import jax
import jax.numpy as jnp
from jax import lax
from jax.experimental import pallas as pl
from jax.experimental.pallas import tpu as pltpu
from jax.experimental.pallas import tpu_sc as plsc

NF = 26
VOCAB = 100000
D = 32
B = 16384
NC, NS, L = 2, 16, 16
NW = NC * NS

CHUNK = 128
N_BCHUNKS = B // CHUNK
TOT_CHUNKS = NF * N_BCHUNKS
CPW = TOT_CHUNKS // NW
OP = CHUNK + 9

RB = 320
PB = RB * D // 128
N_DCH = NF * VOCAB // RB
K1B_ITERS = -(-N_DCH // NW)


def _detile(d2):
    def body(t_hbm, pk_hbm, st0, st1, pack0, pack1, ss0, ss1, ws0, ws1):
        wid = lax.axis_index("s") * NC + lax.axis_index("c")

        def v0_of(i):
            return jnp.minimum(wid + NW * i, N_DCH - 1) * RB

        def fire_st(i, st, sem):
            pltpu.async_copy(
                t_hbm.at[pl.ds(pl.multiple_of(v0_of(i), 8), RB)], st, sem)

        def wait_st(st, sem):
            pltpu.make_async_copy(t_hbm.at[pl.ds(0, RB)], st, sem).wait()

        def repack(st, pack):
            def prow(q2, carry):
                for u in range(2):
                    q = q2 * 2 + u
                    for vi in range(4):
                        for h in range(2):
                            pack[q, pl.ds(vi * D + h * L, L)] = (
                                st[q * 4 + vi, pl.ds(h * L, L)])
                return carry
            lax.fori_loop(0, PB // 2, prow, 0)

        def fire_pack(i, pack, sem):
            r0 = pl.multiple_of(v0_of(i) // 4, 8)
            pltpu.async_copy(pack, pk_hbm.at[pl.ds(r0, PB)], sem)

        def wait_pack(pack, sem):
            pltpu.make_async_copy(pack, pk_hbm.at[pl.ds(0, PB)], sem).wait()

        fire_st(0, st0, ss0)

        def gloop(i, carry):
            even = (i % 2) == 0

            @pl.when(even)
            def _():
                wait_st(st0, ss0)
                @pl.when(i + 1 < K1B_ITERS)
                def _():
                    fire_st(i + 1, st1, ss1)
                @pl.when(i >= 2)
                def _():
                    wait_pack(pack0, ws0)
                repack(st0, pack0)
                fire_pack(i, pack0, ws0)

            @pl.when(jnp.logical_not(even))
            def _():
                wait_st(st1, ss1)
                @pl.when(i + 1 < K1B_ITERS)
                def _():
                    fire_st(i + 1, st0, ss0)
                @pl.when(i >= 2)
                def _():
                    wait_pack(pack1, ws1)
                repack(st1, pack1)
                fire_pack(i, pack1, ws1)

            return carry

        lax.fori_loop(0, K1B_ITERS, gloop, 0)

        wait_pack(pack0, ws0)
        wait_pack(pack1, ws1)

    return pl.kernel(
        body,
        out_type=jax.ShapeDtypeStruct((NF * VOCAB * D // 128, 128),
                                      jnp.float32),
        mesh=plsc.VectorSubcoreMesh(
            core_axis_name="c", subcore_axis_name="s",
            num_cores=NC, num_subcores=NS,
        ),
        scratch_types=[
            pltpu.VMEM((RB, D), jnp.float32),
            pltpu.VMEM((RB, D), jnp.float32),
            pltpu.VMEM((PB, 128), jnp.float32),
            pltpu.VMEM((PB, 128), jnp.float32),
            pltpu.SemaphoreType.DMA,
            pltpu.SemaphoreType.DMA,
            pltpu.SemaphoreType.DMA,
            pltpu.SemaphoreType.DMA,
        ],
        compiler_params=pltpu.CompilerParams(use_tc_tiling_on_sc=True,
                                             needs_layout_passes=False),
    )(d2)


def _gather(pk2d, xf):
    def body(tab, xf_hbm, out_hbm, idx_v, rows0, rows1, o0, o1,
             gs0, gs1, ws0, ws1):
        wid = lax.axis_index("s") * NC + lax.axis_index("c")
        c0 = wid * CPW
        lanes = lax.iota(jnp.int32, L)

        pltpu.sync_copy(xf_hbm.at[pl.ds(c0, CPW)], idx_v)

        def ixrow(cc, carry):
            f = (c0 + cc) // N_BCHUNKS
            for j in range(CHUNK // L):
                sl = pl.ds(j * L, L)
                idx_v[cc, sl] = idx_v[cc, sl] + f * VOCAB
            return carry
        lax.fori_loop(0, CPW, ixrow, 0)

        def fire_gather(cc, rows, sem):
            pltpu.async_copy(tab.at[idx_v.at[cc]], rows, sem)

        def wait_gather(rows, sem):
            pltpu.make_async_copy(tab.at[pl.ds(0, CHUNK)], rows, sem).wait()

        def transpose(rows, o):
            rv0 = lanes
            rv1 = 16 + lanes

            def irow(i8, carry):
                for u in range(8):
                    i = i8 * 8 + u
                    cv = lanes * 0 + i
                    plsc.store_scatter(o, [rv0, cv], rows[i, pl.ds(0, 16)])
                    plsc.store_scatter(o, [rv1, cv], rows[i, pl.ds(16, 16)])
                return carry
            lax.fori_loop(0, CHUNK // 8, irow, 0)

        def fire_writes(cc, o, sem):
            c = c0 + cc
            f = c // N_BCHUNKS
            bt = c - f * N_BCHUNKS
            for dt in range(4):
                pltpu.async_copy(o.at[pl.ds(dt * 8, 8), pl.ds(0, CHUNK)],
                                 out_hbm.at[f, dt, bt], sem)

        def wait_writes(o, sem):
            for dt in range(4):
                pltpu.make_async_copy(o.at[pl.ds(0, 8), pl.ds(0, CHUNK)],
                                      out_hbm.at[0, 0, 0], sem).wait()

        fire_gather(0, rows0, gs0)

        def gloop(cc, carry):
            even = (cc % 2) == 0

            @pl.when(even)
            def _():
                wait_gather(rows0, gs0)
                @pl.when(cc + 1 < CPW)
                def _():
                    fire_gather(cc + 1, rows1, gs1)
                @pl.when(cc >= 2)
                def _():
                    wait_writes(o0, ws0)
                transpose(rows0, o0)
                fire_writes(cc, o0, ws0)

            @pl.when(jnp.logical_not(even))
            def _():
                wait_gather(rows1, gs1)
                @pl.when(cc + 1 < CPW)
                def _():
                    fire_gather(cc + 1, rows0, gs0)
                @pl.when(cc >= 2)
                def _():
                    wait_writes(o1, ws1)
                transpose(rows1, o1)
                fire_writes(cc, o1, ws1)

            return carry

        lax.fori_loop(0, CPW, gloop, 0)

        wait_writes(o0, ws0)
        wait_writes(o1, ws1)

    return pl.kernel(
        body,
        out_type=jax.ShapeDtypeStruct((NF, 4, N_BCHUNKS, 8, 128),
                                      jnp.float32),
        mesh=plsc.VectorSubcoreMesh(
            core_axis_name="c", subcore_axis_name="s",
            num_cores=NC, num_subcores=NS,
        ),
        scratch_types=[
            pltpu.VMEM((CPW, CHUNK), jnp.int32),
            pltpu.VMEM((CHUNK, D), jnp.float32),
            pltpu.VMEM((CHUNK, D), jnp.float32),
            pltpu.VMEM((D, OP), jnp.float32),
            pltpu.VMEM((D, OP), jnp.float32),
            pltpu.SemaphoreType.DMA,
            pltpu.SemaphoreType.DMA,
            pltpu.SemaphoreType.DMA,
            pltpu.SemaphoreType.DMA,
        ],
        compiler_params=pltpu.CompilerParams(use_tc_tiling_on_sc=False,
                                             needs_layout_passes=False),
    )(pk2d, xf)


def kernel(x, tables):
    b, nf = x.shape
    _, vocab, d = tables.shape
    xf = jnp.transpose(x.astype(jnp.int32)).reshape(nf * b // CHUNK, CHUNK)
    pk = _detile(tables.reshape(nf * vocab, d))
    out5 = _gather(pk.reshape(nf * vocab, d), xf)
    return jnp.transpose(out5, (2, 4, 0, 1, 3)).reshape(b, nf, d)
